# baseline (speedup 1.0000x reference)
import numpy as np
import jax
import jax.numpy as jnp
from jax import lax
from jax.experimental import pallas as pl
from jax.experimental.pallas import tpu as pltpu
from jax.experimental.pallas import tpu_sc as plsc

_B = 256
_LMAX = 256
_M = 2 * _LMAX - 1
_K = _LMAX * _LMAX
_NC, _NS, _L = 2, 16, 16
_NW = _NC * _NS
_KS = _K // _NW
_G = _KS // _L


def _build_geometry():
    cols = np.arange(_M)
    starts = np.abs(cols - (_LMAX - 1))
    l_of_k = np.concatenate([np.arange(s, _LMAX) for s in starts])
    c_of_k = np.repeat(cols, _LMAX - starts)
    spans = []
    packed = np.zeros((_NW, _KS), np.int32)
    for s in range(_NW):
        sl = slice(s * _KS, (s + 1) * _KS)
        lk, ck = l_of_k[sl], c_of_k[sl]
        r0, c0 = int(lk.min()), int(ck.min())
        nr = int(lk.max()) - r0 + 1
        c0 = (c0 // 8) * 8
        w = int(ck.max()) - c0 + 1
        w = min(-(-w // 8) * 8, _M - c0)
        spans.append((r0, nr, c0, w))
        packed[s] = ((lk - r0).astype(np.int32) << 16) | (ck - c0).astype(np.int32)
    return spans, packed


_SPANS, _PACKED = _build_geometry()


def _sc_body(tensor_hbm, idx_hbm, out_hbm, idx_v, out_v):
    wid = lax.axis_index("c") * _NS + lax.axis_index("s")
    for s, (r0, nr, c0, w) in enumerate(_SPANS):
        @pl.when(wid == s)
        def _span(s=s, r0=r0, nr=nr, c0=c0, w=w):
            def scoped(slab):
                pltpu.sync_copy(idx_hbm.at[s], idx_v)

                @pl.loop(0, _B)
                def _batch(b):
                    pltpu.sync_copy(
                        tensor_hbm.at[b, pl.ds(r0, nr), pl.ds(c0, w)], slab)

                    @pl.loop(0, _G)
                    def _gather(g):
                        iv = idx_v[pl.ds(g * _L, _L)]
                        rows = lax.shift_right_logical(iv, 16)
                        cls = lax.bitwise_and(iv, jnp.int32(0xFFFF))
                        out_v[pl.ds(g * _L, _L)] = plsc.load_gather(
                            slab, [rows, cls])

                    pltpu.sync_copy(out_v, out_hbm.at[b, pl.ds(s * _KS, _KS)])

            pl.run_scoped(scoped, pltpu.VMEM((nr, w), jnp.float32))


def kernel(tensor):
    idx = jnp.asarray(_PACKED)
    mesh = plsc.VectorSubcoreMesh(core_axis_name="c", subcore_axis_name="s")
    f = pl.kernel(
        _sc_body,
        out_type=jax.ShapeDtypeStruct((_B, _K), jnp.float32),
        mesh=mesh,
        compiler_params=pltpu.CompilerParams(
            use_tc_tiling_on_sc=False, needs_layout_passes=False),
        scratch_types=[
            pltpu.VMEM((_KS,), jnp.int32),
            pltpu.VMEM((_KS,), jnp.float32),
        ],
    )
    return f(tensor, idx)
